# Initial kernel scaffold; baseline (speedup 1.0000x reference)
import jax
import jax.numpy as jnp
from jax.experimental import pallas as pl


def kernel(X, Y, W, rows, cols):
    raise NotImplementedError("write your pallas kernel here")


# speedup vs baseline: 2.4408x; 2.4408x over previous
import functools

import jax
import jax.numpy as jnp
from jax import lax
from jax.experimental import pallas as pl
from jax.experimental.pallas import tpu as pltpu
from jax.experimental.pallas import tpu_sc as plsc

N_NODES = 50000
N_EDGES = 800000
IN1_DIM = 80
IN2_DIM = 4
OUT_DIM = 240
W_NUMEL = 112

NB = 80
CH = 128
T_TASKS = N_NODES // NB
NOFF = 640
E_PAD = N_EDGES + 2 * CH

INV_SQRT3 = 1.0 / (3.0 ** 0.5)
INV_SQRT2 = 1.0 / (2.0 ** 0.5)

NC = 2
NS = 16
NW = NC * NS


def _sc_body(x_hbm, rows_hbm, cols_hbm, y_hbm, w_hbm, offs_hbm, z_hbm,
             acc_v, rows_v, cols_v, y_v, w_v, x_v, offs_v):
    wid = lax.axis_index("s") * NC + lax.axis_index("c")

    pltpu.sync_copy(offs_hbm, offs_v)

    iota = lax.iota(jnp.int32, 16)
    zero16 = jnp.zeros((16,), jnp.float32)
    c_o1a = iota
    c_o1b = iota + 16
    c_o2 = iota + 32
    c_o3a = [3 * iota + (48 + c) for c in range(3)]
    c_o3b = [3 * iota + (96 + c) for c in range(3)]
    c_o4 = [3 * iota + (144 + c) for c in range(3)]
    c_o5 = [3 * iota + (192 + c) for c in range(3)]

    def edge_body(j, carry):
        n0 = carry
        jv = jnp.full((16,), j, jnp.int32)
        r = plsc.load_gather(rows_v, [jv]) - n0

        x0a = plsc.load_gather(x_v, [jv, iota])
        x0b = plsc.load_gather(x_v, [jv, iota + 16])
        x1 = [plsc.load_gather(x_v, [jv, 3 * iota + (32 + c)]) for c in range(3)]

        y0 = plsc.load_gather(y_v, [4 * jv])
        y1 = [plsc.load_gather(y_v, [4 * jv + (1 + c)]) for c in range(3)]

        wbase = 112 * j
        w1a = w_v[pl.ds(wbase, 16)]
        w1b = w_v[pl.ds(wbase + 16, 16)]
        w2 = w_v[pl.ds(wbase + 32, 16)]
        w3a = w_v[pl.ds(wbase + 48, 16)]
        w3b = w_v[pl.ds(wbase + 64, 16)]
        w4 = w_v[pl.ds(wbase + 80, 16)]
        w5 = w_v[pl.ds(wbase + 96, 16)]

        plsc.addupdate_scatter(acc_v, [r, c_o1a], w1a * (x0a * y0))
        plsc.addupdate_scatter(acc_v, [r, c_o1b], w1b * (x0b * y0))
        dot = x1[0] * y1[0] + x1[1] * y1[1] + x1[2] * y1[2]
        plsc.addupdate_scatter(acc_v, [r, c_o2], w2 * (dot * INV_SQRT3))
        t3a = w3a * x0a
        t3b = w3b * x0b
        for c in range(3):
            plsc.addupdate_scatter(acc_v, [r, c_o3a[c]], t3a * y1[c])
            plsc.addupdate_scatter(acc_v, [r, c_o3b[c]], t3b * y1[c])
        t4 = w4 * y0
        for c in range(3):
            plsc.addupdate_scatter(acc_v, [r, c_o4[c]], t4 * x1[c])
        w5s = w5 * INV_SQRT2
        cr = [x1[1] * y1[2] - x1[2] * y1[1],
              x1[2] * y1[0] - x1[0] * y1[2],
              x1[0] * y1[1] - x1[1] * y1[0]]
        for c in range(3):
            plsc.addupdate_scatter(acc_v, [r, c_o5[c]], w5s * cr[c])
        return carry

    def chunk_body(c, carry):
        n0, e0, e1, e0a = carry
        base = e0a + c * CH
        pltpu.sync_copy(rows_hbm.at[pl.ds(base, CH)], rows_v)
        pltpu.sync_copy(cols_hbm.at[pl.ds(base, CH)], cols_v)
        pltpu.sync_copy(y_hbm.at[pl.ds(4 * base, 4 * CH)], y_v)
        pltpu.sync_copy(w_hbm.at[pl.ds(112 * base, 112 * CH)], w_v)
        pltpu.sync_copy(x_hbm.at[cols_v], x_v)
        jlo = jnp.maximum(e0 - base, 0)
        jhi = jnp.minimum(e1 - base, CH)
        lax.fori_loop(jlo, jhi, edge_body, n0, unroll=False)
        return carry

    def zero_body(i, _):
        for k in range(OUT_DIM // 16):
            acc_v[i, pl.ds(16 * k, 16)] = zero16
        return 0

    def task_body(i, _):
        t = wid + i * NW
        n0 = t * NB
        ev = offs_v[pl.ds(t, 16)]
        e0 = ev[0]
        e1 = ev[1]
        e0a = (e0 // 8) * 8
        lax.fori_loop(0, NB, zero_body, 0, unroll=False)
        nchunks = (e1 - e0a + CH - 1) // CH
        lax.fori_loop(0, nchunks, chunk_body, (n0, e0, e1, e0a), unroll=False)
        pltpu.sync_copy(acc_v, z_hbm.at[pl.ds(n0, NB)])
        return 0

    ntasks = (T_TASKS - wid + NW - 1) // NW
    lax.fori_loop(0, ntasks, task_body, 0, unroll=False)


@jax.jit
def _tp_conv(X, Y, W, rows, cols):
    pad = E_PAD - N_EDGES
    rows_p = jnp.concatenate([rows, jnp.zeros((pad,), jnp.int32)])
    cols_p = jnp.concatenate([cols, jnp.zeros((pad,), jnp.int32)])
    y_p = jnp.concatenate([Y.reshape(-1), jnp.zeros((4 * pad,), jnp.float32)])
    w_p = jnp.concatenate([W.reshape(-1), jnp.zeros((112 * pad,), jnp.float32)])
    bounds = jnp.arange(0, NOFF, dtype=jnp.int32) * NB
    offs = jnp.searchsorted(rows, bounds, side="left").astype(jnp.int32)
    offs = jnp.minimum(offs, N_EDGES)

    mesh = plsc.VectorSubcoreMesh(core_axis_name="c", subcore_axis_name="s")
    run = pl.kernel(
        _sc_body,
        out_type=jax.ShapeDtypeStruct((N_NODES, OUT_DIM), jnp.float32),
        mesh=mesh,
        compiler_params=pltpu.CompilerParams(
            needs_layout_passes=False, use_tc_tiling_on_sc=False),
        scratch_types=[
            pltpu.VMEM((NB, OUT_DIM), jnp.float32),
            pltpu.VMEM((CH,), jnp.int32),
            pltpu.VMEM((CH,), jnp.int32),
            pltpu.VMEM((4 * CH,), jnp.float32),
            pltpu.VMEM((112 * CH,), jnp.float32),
            pltpu.VMEM((CH, IN1_DIM), jnp.float32),
            pltpu.VMEM((NOFF,), jnp.int32),
        ],
    )
    return run(X, rows_p, cols_p, y_p, w_p, offs)


def kernel(X, Y, W, rows, cols):
    return _tp_conv(X, Y, W, rows, cols)


# speedup vs baseline: 2.5988x; 1.0647x over previous
import jax
import jax.numpy as jnp
from jax import lax
from jax.experimental import pallas as pl
from jax.experimental.pallas import tpu as pltpu
from jax.experimental.pallas import tpu_sc as plsc

N_NODES = 50000
N_EDGES = 800000
IN1_DIM = 80
OUT_DIM = 240
OUT_PAD = 256
REC = 128

NB = 80
CH = 128
T_TASKS = N_NODES // NB
NOFF = 640
E_PAD = N_EDGES + 2 * CH

INV_SQRT3 = 1.0 / (3.0 ** 0.5)
INV_SQRT2 = 1.0 / (2.0 ** 0.5)

NC = 2
NS = 16
NW = NC * NS


def _sc_body(x_hbm, rows_hbm, cols_hbm, w_hbm, offs_hbm, z_hbm,
             acc_v, rows_v, cols_v, w_v, x_v, offs_v):
    wid = lax.axis_index("s") * NC + lax.axis_index("c")

    pltpu.sync_copy(offs_hbm, offs_v)

    iota = lax.iota(jnp.int32, 16)
    zero16 = jnp.zeros((16,), jnp.float32)
    c_o1a = iota
    c_o1b = iota + 16
    c_o2 = iota + 32
    c_o3a = [3 * iota + (48 + c) for c in range(3)]
    c_o3b = [3 * iota + (96 + c) for c in range(3)]
    c_o4 = [3 * iota + (144 + c) for c in range(3)]
    c_o5 = [3 * iota + (192 + c) for c in range(3)]

    def edge_body(j, carry):
        n0 = carry
        jv = jnp.full((16,), j, jnp.int32)
        r = plsc.load_gather(rows_v, [jv]) - n0

        x0a = plsc.load_gather(x_v, [jv, iota])
        x0b = plsc.load_gather(x_v, [jv, iota + 16])
        x1 = [plsc.load_gather(x_v, [jv, 3 * iota + (32 + c)]) for c in range(3)]

        y0 = plsc.load_gather(w_v, [jv, jnp.full((16,), 112, jnp.int32)])
        y1 = [plsc.load_gather(w_v, [jv, jnp.full((16,), 113 + c, jnp.int32)])
              for c in range(3)]

        w1a = plsc.load_gather(w_v, [jv, iota])
        w1b = plsc.load_gather(w_v, [jv, iota + 16])
        w2 = plsc.load_gather(w_v, [jv, iota + 32])
        w3a = plsc.load_gather(w_v, [jv, iota + 48])
        w3b = plsc.load_gather(w_v, [jv, iota + 64])
        w4 = plsc.load_gather(w_v, [jv, iota + 80])
        w5 = plsc.load_gather(w_v, [jv, iota + 96])

        plsc.addupdate_scatter(acc_v, [r, c_o1a], w1a * (x0a * y0))
        plsc.addupdate_scatter(acc_v, [r, c_o1b], w1b * (x0b * y0))
        dot = x1[0] * y1[0] + x1[1] * y1[1] + x1[2] * y1[2]
        plsc.addupdate_scatter(acc_v, [r, c_o2], w2 * (dot * INV_SQRT3))
        t3a = w3a * x0a
        t3b = w3b * x0b
        for c in range(3):
            plsc.addupdate_scatter(acc_v, [r, c_o3a[c]], t3a * y1[c])
            plsc.addupdate_scatter(acc_v, [r, c_o3b[c]], t3b * y1[c])
        t4 = w4 * y0
        for c in range(3):
            plsc.addupdate_scatter(acc_v, [r, c_o4[c]], t4 * x1[c])
        w5s = w5 * INV_SQRT2
        cr = [x1[1] * y1[2] - x1[2] * y1[1],
              x1[2] * y1[0] - x1[0] * y1[2],
              x1[0] * y1[1] - x1[1] * y1[0]]
        for c in range(3):
            plsc.addupdate_scatter(acc_v, [r, c_o5[c]], w5s * cr[c])
        return carry

    def chunk_body(c, carry):
        n0, e0, e1, e0a = carry
        base = e0a + c * CH
        pltpu.sync_copy(rows_hbm.at[pl.ds(base, CH)], rows_v)
        pltpu.sync_copy(cols_hbm.at[pl.ds(base, CH)], cols_v)
        pltpu.sync_copy(w_hbm.at[pl.ds(base, CH)], w_v)
        pltpu.sync_copy(x_hbm.at[cols_v], x_v)
        jlo = jnp.maximum(e0 - base, 0)
        jhi = jnp.minimum(e1 - base, CH)
        lax.fori_loop(jlo, jhi, edge_body, n0, unroll=False)
        return carry

    def zero_body(i, _):
        for k in range(OUT_PAD // 16):
            acc_v[i, pl.ds(16 * k, 16)] = zero16
        return 0

    def task_body(i, _):
        t = wid + i * NW
        n0 = t * NB
        ev = offs_v[pl.ds(t, 16)]
        e0 = ev[0]
        e1 = ev[1]
        e0a = (e0 // 8) * 8
        lax.fori_loop(0, NB, zero_body, 0, unroll=False)
        nchunks = (e1 - e0a + CH - 1) // CH
        lax.fori_loop(0, nchunks, chunk_body, (n0, e0, e1, e0a), unroll=False)
        pltpu.sync_copy(acc_v, z_hbm.at[pl.ds(n0, NB)])
        return 0

    ntasks = (T_TASKS - wid + NW - 1) // NW
    lax.fori_loop(0, ntasks, task_body, 0, unroll=False)


@jax.jit
def _tp_conv(X, Y, W, rows, cols):
    pad = E_PAD - N_EDGES
    rows_p = jnp.concatenate([rows, jnp.zeros((pad,), jnp.int32)])
    cols_p = jnp.concatenate([cols, jnp.zeros((pad,), jnp.int32)])
    w_p = jnp.concatenate(
        [W, Y, jnp.zeros((N_EDGES, REC - 116), jnp.float32)], axis=1)
    w_p = jnp.concatenate([w_p, jnp.zeros((pad, REC), jnp.float32)], axis=0)
    x_p = jnp.concatenate(
        [X, jnp.zeros((N_NODES, REC - IN1_DIM), jnp.float32)], axis=1)
    bounds = jnp.arange(0, NOFF, dtype=jnp.int32) * NB
    offs = jnp.searchsorted(rows, bounds, side="left").astype(jnp.int32)
    offs = jnp.minimum(offs, N_EDGES)

    mesh = plsc.VectorSubcoreMesh(core_axis_name="c", subcore_axis_name="s")
    run = pl.kernel(
        _sc_body,
        out_type=jax.ShapeDtypeStruct((N_NODES, OUT_PAD), jnp.float32),
        mesh=mesh,
        compiler_params=pltpu.CompilerParams(
            needs_layout_passes=False, use_tc_tiling_on_sc=False),
        scratch_types=[
            pltpu.VMEM((NB, OUT_PAD), jnp.float32),
            pltpu.VMEM((CH,), jnp.int32),
            pltpu.VMEM((CH,), jnp.int32),
            pltpu.VMEM((CH, REC), jnp.float32),
            pltpu.VMEM((CH, REC), jnp.float32),
            pltpu.VMEM((NOFF,), jnp.int32),
        ],
    )
    z = run(x_p, rows_p, cols_p, w_p, offs)
    return z[:, :OUT_DIM]


def kernel(X, Y, W, rows, cols):
    return _tp_conv(X, Y, W, rows, cols)
